# baseline (speedup 1.0000x reference)
import functools

import jax
import jax.numpy as jnp
import numpy as np
from jax import lax
from jax.experimental import pallas as pl
from jax.experimental.pallas import tpu as pltpu
from jax.experimental.pallas import tpu_sc as plsc

H = 128
HM = 153
N = 10000
E = 160000
F32 = jnp.float32

NCORES = 2
NSUB = 16
NW = NCORES * NSUB
CHUNK = 128
NJ = 80
NJC = NJ * CHUNK
E2 = NSUB * NJC
NP = 10240
RPT = NP // NSUB
BR = 1024

_R = np.arange(E2 - E, dtype=np.int32)
_DUMMY = np.stack([N + _R % (NP - N), (_R * 67) % NP]).astype(np.int32)


def _inv_sqrt(d):
    safe = jnp.where(d > 0, d, 1.0)
    return jnp.where(d > 0, lax.rsqrt(safe), 0.0)


def _mlp_pair_body(rank1, x_ref, deg_ref, W1_ref, b1_ref, W2_ref, b2_ref,
                   W3_ref, b3_ref, W0_ref, oa_ref, ob_ref):
    dsi = _inv_sqrt(deg_ref[...])
    x = x_ref[...]
    for k, out in ((0, oa_ref), (1, ob_ref)):
        if rank1:
            w1e = jnp.dot(W0_ref[...], W1_ref[k], preferred_element_type=F32)
            h = x * w1e + b1_ref[k]
        else:
            h = jnp.dot(x, W1_ref[k], preferred_element_type=F32) + b1_ref[k]
        h = jnp.maximum(h, 0.0)
        h = jnp.maximum(jnp.dot(h, W2_ref[k], preferred_element_type=F32) + b2_ref[k], 0.0)
        h = jnp.maximum(jnp.dot(h, W3_ref[k], preferred_element_type=F32) + b3_ref[k], 0.0)
        out[...] = h * dsi


def _mlp_pair(x, deg, W0, W1, b1, W2, b2, W3, b3, rank1):
    grid = (NP // BR,)
    full = lambda *s: pl.BlockSpec(s, lambda i: (0,) * len(s))
    return pl.pallas_call(
        functools.partial(_mlp_pair_body, rank1),
        grid=grid,
        in_specs=[
            pl.BlockSpec((BR, 1 if rank1 else H), lambda i: (i, 0)),
            pl.BlockSpec((BR, 1), lambda i: (i, 0)),
            full(2, H, HM), full(2, 1, HM),
            full(2, HM, HM), full(2, 1, HM),
            full(2, HM, H), full(2, 1, H),
            full(1, H),
        ],
        out_specs=[pl.BlockSpec((BR, H), lambda i: (i, 0))] * 2,
        out_shape=[jax.ShapeDtypeStruct((NP, H), F32)] * 2,
    )(x, deg, W1, b1, W2, b2, W3, b3, W0)


def _combine_body(rank1, pp_ref, pn_ref, deg_ref, prev_ref, w0_ref, W_ref,
                  b_ref, out_ref):
    dti = _inv_sqrt(deg_ref[...])
    a = pp_ref[...] * dti
    c = pn_ref[...] * dti
    acc = jnp.dot(a, W_ref[0], preferred_element_type=F32)
    acc += jnp.dot(c, W_ref[1], preferred_element_type=F32)
    if rank1:
        we = jnp.dot(w0_ref[...], W_ref[2], preferred_element_type=F32)
        acc += prev_ref[...] * we
    else:
        acc += jnp.dot(prev_ref[...], W_ref[2], preferred_element_type=F32)
    out_ref[...] = acc + b_ref[...]


def _combine(pp, pn, deg, prev, W3s, b, rank1, w0=None, rows=NP):
    grid = (10,)
    br = rows // 10
    full = lambda *s: pl.BlockSpec(s, lambda i: (0,) * len(s))
    if w0 is None:
        w0 = jnp.zeros((1, H), F32)
    return pl.pallas_call(
        functools.partial(_combine_body, rank1),
        grid=grid,
        in_specs=[
            pl.BlockSpec((br, H), lambda i: (i, 0)),
            pl.BlockSpec((br, H), lambda i: (i, 0)),
            pl.BlockSpec((br, 1), lambda i: (i, 0)),
            pl.BlockSpec((br, 1 if rank1 else H), lambda i: (i, 0)),
            full(1, H), full(3, H, H), full(1, H),
        ],
        out_specs=pl.BlockSpec((br, H), lambda i: (i, 0)),
        out_shape=jax.ShapeDtypeStruct((rows, H), F32),
    )(pp, pn, deg, prev, w0, W3s, b)


def _conv2_body(z0, s0, t0, z1, s1, t1, zrows, out0, out1,
                acc, sbuf, tbuf, rows, semg, semt, sems):
    cid = lax.axis_index("c")
    sid = lax.axis_index("s")
    base = sid * RPT
    ebase = sid * NJC
    for k, (z, s, t, out) in enumerate(((z0, s0, t0, out0),
                                        (z1, s1, t1, out1))):
        @pl.when(cid == k)
        def _():
            pltpu.sync_copy(s.at[pl.ds(ebase, NJC)], sbuf)
            pltpu.sync_copy(zrows, acc.at[pl.ds(base, RPT)])
            plsc.subcore_barrier()

            def gat(j, b):
                pltpu.async_copy(z.at[sbuf.at[pl.ds(j * CHUNK, CHUNK)]],
                                 rows.at[b], semg[b])

            def tcp(j, b):
                pltpu.async_copy(t.at[pl.ds(ebase + j * CHUNK, CHUNK)],
                                 tbuf.at[b], semt[b])

            def wg(b):
                pltpu.make_async_copy(z.at[sbuf.at[pl.ds(0, CHUNK)]],
                                      rows.at[b], semg[b]).wait()
                pltpu.make_async_copy(t.at[pl.ds(0, CHUNK)],
                                      tbuf.at[b], semt[b]).wait()

            def sca(b):
                pltpu.async_copy(rows.at[b], acc.at[tbuf.at[b]], sems[b],
                                 add=True)

            def ws(b):
                pltpu.make_async_copy(rows.at[b], acc.at[tbuf.at[b]],
                                      sems[b]).wait()

            gat(0, 0); tcp(0, 0)
            wg(0); sca(0)
            gat(1, 1); tcp(1, 1)

            @pl.loop(0, (NJ - 2) // 2)
            def _(g):
                for i in (0, 1):
                    c = 1 + g * 2 + i
                    b = (1 + i) % 2
                    wg(b)
                    sca(b)
                    ws(1 - b)
                    gat(c + 1, 1 - b)
                    tcp(c + 1, 1 - b)

            wg(1); sca(1)
            ws(0); ws(1)

            plsc.subcore_barrier()
            pltpu.sync_copy(acc.at[pl.ds(base, RPT)],
                            out.at[pl.ds(base, RPT)])


@functools.cache
def _conv2_kernel():
    mesh = plsc.VectorSubcoreMesh(core_axis_name="c", subcore_axis_name="s")
    return pl.kernel(
        _conv2_body,
        mesh=mesh,
        out_type=[jax.ShapeDtypeStruct((NP, H), F32)] * 2,
        scratch_types=[
            pltpu.VMEM_SHARED((NP, H), F32),
            pltpu.VMEM((NJC,), jnp.int32),
            pltpu.VMEM((2, CHUNK), jnp.int32),
            pltpu.VMEM((2, CHUNK, H), F32),
            [pltpu.SemaphoreType.DMA] * 2,
            [pltpu.SemaphoreType.DMA] * 2,
            [pltpu.SemaphoreType.DMA] * 2,
        ],
    )


def _conv_pair(z0, ei0, z1, ei1, zrows):
    return _conv2_kernel()(z0, ei0[0], ei0[1], z1, ei1[0], ei1[1], zrows)


def _pad_edges(ei):
    return jnp.concatenate([ei, jnp.asarray(_DUMMY)], axis=1)


def kernel(x_clause, x_variable, deg_clause, deg_variable, ei_cp, ei_cn,
           ei_rp, ei_rn, W0c, W0v, conv_ls_W, conv_ls_b, mlp_W1, mlp_b1,
           mlp_W2, mlp_b2, mlp_W3, mlp_b3, lins_c_W, lins_c_b, lins_v_W,
           lins_v_b):
    del conv_ls_W, conv_ls_b
    pad = NP - N
    xc = jnp.pad(x_clause, ((0, pad), (0, 0)))
    xv = jnp.pad(x_variable, ((0, pad), (0, 0)))
    degc = jnp.pad(deg_clause.reshape(N, 1), ((0, pad), (0, 0)))
    degv = jnp.pad(deg_variable.reshape(N, 1), ((0, pad), (0, 0)))
    e_cp, e_cn = _pad_edges(ei_cp), _pad_edges(ei_cn)
    e_rp, e_rn = _pad_edges(ei_rp), _pad_edges(ei_rn)
    zrows = jnp.zeros((RPT, H), F32)

    def mw(l, r0):
        return (mlp_W1[l, r0:r0 + 2], mlp_b1[l, r0:r0 + 2, None],
                mlp_W2[l, r0:r0 + 2], mlp_b2[l, r0:r0 + 2, None],
                mlp_W3[l, r0:r0 + 2], mlp_b3[l, r0:r0 + 2, None])

    zv0, zv1 = _mlp_pair(xv, degv, W0v, *mw(0, 2), rank1=True)
    zc0, zc1 = _mlp_pair(xc, degc, W0c, *mw(0, 0), rank1=True)

    pcp, pcn = _conv_pair(zv0, e_rp, zv1, e_rn, zrows)
    pvp, pvn = _conv_pair(zc0, e_cp, zc1, e_cn, zrows)

    xc1 = _combine(pcp, pcn, degc, xc, lins_c_W[0].reshape(3, H, H),
                   lins_c_b[0][None], rank1=True, w0=W0c)
    xv1 = _combine(pvp, pvn, degv, xv, lins_v_W[0].reshape(3, H, H),
                   lins_v_b[0][None], rank1=True, w0=W0v)

    zq0, zq1 = _mlp_pair(xc1, degc, W0c, *mw(1, 0), rank1=False)
    qvp, qvn = _conv_pair(zq0, e_cp, zq1, e_cn, zrows)

    return _combine(qvp, qvn, degv, xv1, lins_v_W[1].reshape(3, H, H),
                    lins_v_b[1][None], rank1=False, rows=N)


# speedup vs baseline: 1.1457x; 1.1457x over previous
import functools

import jax
import jax.numpy as jnp
import numpy as np
from jax import lax
from jax.experimental import pallas as pl
from jax.experimental.pallas import tpu as pltpu
from jax.experimental.pallas import tpu_sc as plsc

H = 128
HM = 153
N = 10000
E = 160000
F32 = jnp.float32

NCORES = 2
NSUB = 16
NW = NCORES * NSUB
CHUNK = 128
NJ = 80
NJC = NJ * CHUNK
E2 = NSUB * NJC
NP = 10240
RPT = NP // NSUB
BR = 1024

_R = np.arange(E2 - E, dtype=np.int32)
_DUMMY = np.stack([N + _R % (NP - N), (_R * 67) % NP]).astype(np.int32)


def _inv_sqrt(d):
    safe = jnp.where(d > 0, d, 1.0)
    return jnp.where(d > 0, lax.rsqrt(safe), 0.0)


def _mlp_pair_body(rank1, x_ref, deg_ref, W1_ref, b1_ref, W2_ref, b2_ref,
                   W3_ref, b3_ref, W0_ref, oa_ref, ob_ref):
    dsi = _inv_sqrt(deg_ref[...])
    x = x_ref[...]
    for k, out in ((0, oa_ref), (1, ob_ref)):
        if rank1:
            w1e = jnp.dot(W0_ref[...], W1_ref[k], preferred_element_type=F32)
            h = x * w1e + b1_ref[k]
        else:
            h = jnp.dot(x, W1_ref[k], preferred_element_type=F32) + b1_ref[k]
        h = jnp.maximum(h, 0.0)
        h = jnp.maximum(jnp.dot(h, W2_ref[k], preferred_element_type=F32) + b2_ref[k], 0.0)
        h = jnp.maximum(jnp.dot(h, W3_ref[k], preferred_element_type=F32) + b3_ref[k], 0.0)
        out[...] = h * dsi


def _mlp_pair(x, deg, W0, W1, b1, W2, b2, W3, b3, rank1):
    grid = (NP // BR,)
    full = lambda *s: pl.BlockSpec(s, lambda i: (0,) * len(s))
    return pl.pallas_call(
        functools.partial(_mlp_pair_body, rank1),
        grid=grid,
        in_specs=[
            pl.BlockSpec((BR, 1 if rank1 else H), lambda i: (i, 0)),
            pl.BlockSpec((BR, 1), lambda i: (i, 0)),
            full(2, H, HM), full(2, 1, HM),
            full(2, HM, HM), full(2, 1, HM),
            full(2, HM, H), full(2, 1, H),
            full(1, H),
        ],
        out_specs=[pl.BlockSpec((BR, H), lambda i: (i, 0))] * 2,
        out_shape=[jax.ShapeDtypeStruct((NP, H), F32)] * 2,
    )(x, deg, W1, b1, W2, b2, W3, b3, W0)


def _combine_body(rank1, pp_ref, pn_ref, deg_ref, prev_ref, w0_ref, W_ref,
                  b_ref, out_ref):
    dti = _inv_sqrt(deg_ref[...])
    a = pp_ref[...] * dti
    c = pn_ref[...] * dti
    acc = jnp.dot(a, W_ref[0], preferred_element_type=F32)
    acc += jnp.dot(c, W_ref[1], preferred_element_type=F32)
    if rank1:
        we = jnp.dot(w0_ref[...], W_ref[2], preferred_element_type=F32)
        acc += prev_ref[...] * we
    else:
        acc += jnp.dot(prev_ref[...], W_ref[2], preferred_element_type=F32)
    out_ref[...] = acc + b_ref[...]


def _combine(pp, pn, deg, prev, W3s, b, rank1, w0=None, rows=NP):
    grid = (10,)
    br = rows // 10
    full = lambda *s: pl.BlockSpec(s, lambda i: (0,) * len(s))
    if w0 is None:
        w0 = jnp.zeros((1, H), F32)
    return pl.pallas_call(
        functools.partial(_combine_body, rank1),
        grid=grid,
        in_specs=[
            pl.BlockSpec((br, H), lambda i: (i, 0)),
            pl.BlockSpec((br, H), lambda i: (i, 0)),
            pl.BlockSpec((br, 1), lambda i: (i, 0)),
            pl.BlockSpec((br, 1 if rank1 else H), lambda i: (i, 0)),
            full(1, H), full(3, H, H), full(1, H),
        ],
        out_specs=pl.BlockSpec((br, H), lambda i: (i, 0)),
        out_shape=jax.ShapeDtypeStruct((rows, H), F32),
    )(pp, pn, deg, prev, w0, W3s, b)


def _conv2_body(z0, s0, t0, z1, s1, t1, zrows, out0, out1,
                acc, sbuf, tbuf, rows, semg, semt, sems):
    cid = lax.axis_index("c")
    sid = lax.axis_index("s")
    base = sid * RPT
    ebase = sid * NJC
    for k, (z, s, t, out) in enumerate(((z0, s0, t0, out0),
                                        (z1, s1, t1, out1))):
        @pl.when(cid == k)
        def _():
            pltpu.sync_copy(s.at[pl.ds(ebase, NJC)], sbuf)
            pltpu.sync_copy(zrows, acc.at[pl.ds(base, RPT)])
            plsc.subcore_barrier()

            def gat(j, b):
                pltpu.async_copy(z.at[sbuf.at[pl.ds(j * CHUNK, CHUNK)]],
                                 rows.at[b], semg[b])

            def tcp(j, b):
                pltpu.async_copy(t.at[pl.ds(ebase + j * CHUNK, CHUNK)],
                                 tbuf.at[b], semt[b])

            def wg(b):
                pltpu.make_async_copy(z.at[sbuf.at[pl.ds(0, CHUNK)]],
                                      rows.at[b], semg[b]).wait()
                pltpu.make_async_copy(t.at[pl.ds(0, CHUNK)],
                                      tbuf.at[b], semt[b]).wait()

            def sca(b):
                pltpu.async_copy(rows.at[b], acc.at[tbuf.at[b]], sems[b],
                                 add=True)

            def ws(b):
                pltpu.make_async_copy(rows.at[b], acc.at[tbuf.at[b]],
                                      sems[b]).wait()

            gat(0, 0); tcp(0, 0)

            @pl.loop(0, NJ // 2)
            def _(g):
                for b in (0, 1):
                    j = g * 2 + b
                    jn = jnp.minimum(j + 1, NJ - 1)
                    gat(jn, 1 - b)
                    tcp(jn, 1 - b)
                    wg(b)
                    pltpu.sync_copy(rows.at[b], acc.at[tbuf.at[b]], add=True)

            wg(0)

            plsc.subcore_barrier()
            pltpu.sync_copy(acc.at[pl.ds(base, RPT)],
                            out.at[pl.ds(base, RPT)])


@functools.cache
def _conv2_kernel():
    mesh = plsc.VectorSubcoreMesh(core_axis_name="c", subcore_axis_name="s")
    return pl.kernel(
        _conv2_body,
        mesh=mesh,
        out_type=[jax.ShapeDtypeStruct((NP, H), F32)] * 2,
        scratch_types=[
            pltpu.VMEM_SHARED((NP, H), F32),
            pltpu.VMEM((NJC,), jnp.int32),
            pltpu.VMEM((2, CHUNK), jnp.int32),
            pltpu.VMEM((2, CHUNK, H), F32),
            [pltpu.SemaphoreType.DMA] * 2,
            [pltpu.SemaphoreType.DMA] * 2,
            [pltpu.SemaphoreType.DMA] * 2,
        ],
    )


def _conv_pair(z0, ei0, z1, ei1, zrows):
    return _conv2_kernel()(z0, ei0[0], ei0[1], z1, ei1[0], ei1[1], zrows)


def _pad_edges(ei):
    return jnp.concatenate([ei, jnp.asarray(_DUMMY)], axis=1)


def kernel(x_clause, x_variable, deg_clause, deg_variable, ei_cp, ei_cn,
           ei_rp, ei_rn, W0c, W0v, conv_ls_W, conv_ls_b, mlp_W1, mlp_b1,
           mlp_W2, mlp_b2, mlp_W3, mlp_b3, lins_c_W, lins_c_b, lins_v_W,
           lins_v_b):
    del conv_ls_W, conv_ls_b
    pad = NP - N
    xc = jnp.pad(x_clause, ((0, pad), (0, 0)))
    xv = jnp.pad(x_variable, ((0, pad), (0, 0)))
    degc = jnp.pad(deg_clause.reshape(N, 1), ((0, pad), (0, 0)))
    degv = jnp.pad(deg_variable.reshape(N, 1), ((0, pad), (0, 0)))
    e_cp, e_cn = _pad_edges(ei_cp), _pad_edges(ei_cn)
    e_rp, e_rn = _pad_edges(ei_rp), _pad_edges(ei_rn)
    zrows = jnp.zeros((RPT, H), F32)

    def mw(l, r0):
        return (mlp_W1[l, r0:r0 + 2], mlp_b1[l, r0:r0 + 2, None],
                mlp_W2[l, r0:r0 + 2], mlp_b2[l, r0:r0 + 2, None],
                mlp_W3[l, r0:r0 + 2], mlp_b3[l, r0:r0 + 2, None])

    zv0, zv1 = _mlp_pair(xv, degv, W0v, *mw(0, 2), rank1=True)
    zc0, zc1 = _mlp_pair(xc, degc, W0c, *mw(0, 0), rank1=True)

    pcp, pcn = _conv_pair(zv0, e_rp, zv1, e_rn, zrows)
    pvp, pvn = _conv_pair(zc0, e_cp, zc1, e_cn, zrows)

    xc1 = _combine(pcp, pcn, degc, xc, lins_c_W[0].reshape(3, H, H),
                   lins_c_b[0][None], rank1=True, w0=W0c)
    xv1 = _combine(pvp, pvn, degv, xv, lins_v_W[0].reshape(3, H, H),
                   lins_v_b[0][None], rank1=True, w0=W0v)

    zq0, zq1 = _mlp_pair(xc1, degc, W0c, *mw(1, 0), rank1=False)
    qvp, qvn = _conv_pair(zq0, e_cp, zq1, e_cn, zrows)

    return _combine(qvp, qvn, degv, xv1, lins_v_W[1].reshape(3, H, H),
                    lins_v_b[1][None], rank1=False, rows=N)
